# baseline (speedup 1.0000x reference)
import jax
import jax.numpy as jnp
from jax.experimental import pallas as pl
from jax.experimental.pallas import tpu as pltpu

N = 10000
E = 160000
DT = 1.0

NODE_BLOCK = 1000
EDGE_BLOCK = 8000


def _node_mlp_kernel(x_ref, w1_ref, b1_ref, w2_ref, b2_ref, wd_ref, bd_ref, o_ref):
    x = x_ref[...]
    h = jnp.dot(x, w1_ref[...], preferred_element_type=jnp.float32) + b1_ref[...]
    h = h * jax.nn.sigmoid(h)
    h = jnp.dot(h, w2_ref[...], preferred_element_type=jnp.float32) + b2_ref[...]
    h = h * jax.nn.sigmoid(h)
    o_ref[...] = jnp.dot(h, wd_ref[...], preferred_element_type=jnp.float32) + bd_ref[...]


def _edge_kernel(e_ref, w_ref, b_ref, o_ref):
    o_ref[...] = jnp.dot(e_ref[...], w_ref[...], preferred_element_type=jnp.float32) + b_ref[...]


def kernel(nodes, edges, senders, receivers, globals_, W_enc_n, b_enc_n, W_enc_e, b_enc_e, W1, b1, W2, b2, Wd_n, bd_n, Wd_e, bd_e):
    d_feat = nodes.shape[1]
    latent = W1.shape[1]
    node_out = Wd_n.shape[1]
    d_edge = edges.shape[1]
    edge_out = Wd_e.shape[1]

    whole = lambda *shape: pl.BlockSpec(shape, lambda i: (0,) * len(shape))

    out_nodes = pl.pallas_call(
        _node_mlp_kernel,
        grid=(N // NODE_BLOCK,),
        in_specs=[
            pl.BlockSpec((NODE_BLOCK, d_feat), lambda i: (i, 0)),
            whole(d_feat, latent),
            whole(1, latent),
            whole(latent, latent),
            whole(1, latent),
            whole(latent, node_out),
            whole(1, node_out),
        ],
        out_specs=pl.BlockSpec((NODE_BLOCK, node_out), lambda i: (i, 0)),
        out_shape=jax.ShapeDtypeStruct((N, node_out), jnp.float32),
        compiler_params=pltpu.CompilerParams(
            dimension_semantics=("parallel",),
        ),
    )(nodes, W1, b1.reshape(1, -1), W2, b2.reshape(1, -1), Wd_n, bd_n.reshape(1, -1))

    out_edges = pl.pallas_call(
        _edge_kernel,
        grid=(E // EDGE_BLOCK,),
        in_specs=[
            pl.BlockSpec((EDGE_BLOCK, d_edge), lambda i: (i, 0)),
            whole(d_edge, edge_out),
            whole(1, edge_out),
        ],
        out_specs=pl.BlockSpec((EDGE_BLOCK, edge_out), lambda i: (i, 0)),
        out_shape=jax.ShapeDtypeStruct((E, edge_out), jnp.float32),
        compiler_params=pltpu.CompilerParams(
            dimension_semantics=("parallel",),
        ),
    )(edges, Wd_e, bd_e.reshape(1, -1))

    new_globals = globals_ + DT
    return out_nodes, out_edges, new_globals
